# baseline (speedup 1.0000x reference)
import functools
import jax
import jax.numpy as jnp
from jax import lax
from jax.experimental import pallas as pl
from jax.experimental.pallas import tpu as pltpu
from jax.experimental.pallas import tpu_sc as plsc

N = 10000
E = 320000
D_IN = 128
D_HID = 256
NCLS = 40

NC, NS, L = 2, 16, 16
NW = NC * NS
CH = 128
NBUF = 4
K1 = 160
K2 = 80
EPAD = NS * K1 * CH
NPAD = 10112
RPT = NPAD // NS
DC = 80
DEGC = 48
DW2 = 48


def _pipeline(table_hbm, shared, src_v, dst_v, rows, gsem, ssem, n_chunks):
  def g_issue(b, j):
    pltpu.async_copy(table_hbm.at[src_v.at[j]], rows[b], gsem[b])

  def g_wait(b):
    pltpu.make_async_copy(table_hbm.at[src_v.at[0]], rows[b], gsem[b]).wait()

  def s_issue(b, j):
    pltpu.async_copy(rows[b], shared.at[dst_v.at[j]], ssem[b], add=True)

  def s_wait(b):
    pltpu.make_async_copy(rows[b], shared.at[dst_v.at[0]], ssem[b]).wait()

  for b in range(NBUF):
    g_issue(b, b)

  def body(jj, _):
    base_j = jj * NBUF
    for b in range(NBUF):
      g_wait(b)
      s_issue(b, base_j + b)
    for b in range(NBUF):
      s_wait(b)
      g_issue(b, base_j + NBUF + b)
    return _
  lax.fori_loop(0, n_chunks // NBUF - 1, body, 0)

  for b in range(NBUF):
    g_wait(b)
    s_issue(b, n_chunks - NBUF + b)
  for b in range(NBUF):
    s_wait(b)


def _zero_shared_slice(rows0, shared, base, D):
  def zrow(r, _):
    def zcol(q, __):
      rows0[r, pl.ds(q * L, L)] = jnp.zeros((L,), jnp.float32)
      return __
    return lax.fori_loop(0, D // L, zcol, _, unroll=True)
  lax.fori_loop(0, CH, zrow, 0)
  for t in range(RPT // CH):
    pltpu.sync_copy(rows0, shared.at[pl.ds(base + t * CH, CH)])
  rem = RPT % CH
  if rem:
    pltpu.sync_copy(rows0.at[pl.ds(0, rem)],
                    shared.at[pl.ds(base + (RPT // CH) * CH, rem)])


def _mesh():
  return plsc.VectorSubcoreMesh(core_axis_name="c", subcore_axis_name="s",
                                num_cores=NC, num_subcores=NS)


@functools.lru_cache(maxsize=None)
def _seg_l1():
  @functools.partial(
      pl.kernel,
      out_type=jax.ShapeDtypeStruct((NC, NPAD, DC), jnp.float32),
      mesh=_mesh(),
      scratch_types=(
          [pltpu.VMEM((K1 // 2, CH), jnp.int32),
           pltpu.VMEM((K1 // 2, CH), jnp.int32)]
          + [pltpu.VMEM((CH, DC), jnp.float32) for _ in range(NBUF)]
          + [pltpu.SemaphoreType.DMA for _ in range(2 * NBUF)]
          + [pltpu.VMEM_SHARED((NPAD, DC), jnp.float32)]
      ),
      compiler_params=pltpu.CompilerParams(use_tc_tiling_on_sc=False),
  )
  def seg(table_hbm, src_hbm, dst_hbm, out_hbm, src_v, dst_v, *rest):
    rows = rest[:NBUF]
    gsem = rest[NBUF:2 * NBUF]
    ssem = rest[2 * NBUF:3 * NBUF]
    shared = rest[3 * NBUF]
    c = lax.axis_index("c")
    s = lax.axis_index("s")

    base = s * RPT
    K1H = K1 // 2
    for h in range(2):
      ig = pltpu.async_copy(src_hbm.at[c, s, pl.ds(h * K1H, K1H)],
                            src_v, gsem[0])
      ig2 = pltpu.async_copy(dst_hbm.at[s, pl.ds(h * K1H, K1H)],
                             dst_v, gsem[1])
      if h == 0:
        _zero_shared_slice(rows[0], shared, base, DC)
      ig.wait()
      ig2.wait()
      if h == 0:
        plsc.subcore_barrier()
      _pipeline(table_hbm, shared, src_v, dst_v, rows, gsem, ssem, K1H)

    plsc.subcore_barrier()
    pltpu.sync_copy(shared.at[pl.ds(base, RPT)],
                    out_hbm.at[c, pl.ds(base, RPT)])

  return seg


@functools.lru_cache(maxsize=None)
def _seg_l2():
  @functools.partial(
      pl.kernel,
      out_type=jax.ShapeDtypeStruct((NC, NPAD, DW2), jnp.float32),
      mesh=_mesh(),
      scratch_types=(
          [pltpu.VMEM((K2, CH), jnp.int32),
           pltpu.VMEM((K2, CH), jnp.int32)]
          + [pltpu.VMEM((CH, DW2), jnp.float32) for _ in range(NBUF)]
          + [pltpu.SemaphoreType.DMA for _ in range(2 * NBUF)]
          + [pltpu.VMEM_SHARED((NPAD, DW2), jnp.float32)]
      ),
      compiler_params=pltpu.CompilerParams(use_tc_tiling_on_sc=False),
  )
  def seg(table_hbm, src_hbm, dst_hbm, out_hbm, src_v, dst_v, *rest):
    rows = rest[:NBUF]
    gsem = rest[NBUF:2 * NBUF]
    ssem = rest[2 * NBUF:3 * NBUF]
    shared = rest[3 * NBUF]
    c = lax.axis_index("c")
    s = lax.axis_index("s")
    wid = s * NC + c

    ig = pltpu.async_copy(src_hbm.at[wid], src_v, gsem[0])
    ig2 = pltpu.async_copy(dst_hbm.at[wid], dst_v, gsem[1])
    base = s * RPT
    _zero_shared_slice(rows[0], shared, base, DW2)
    ig.wait()
    ig2.wait()
    plsc.subcore_barrier()

    _pipeline(table_hbm, shared, src_v, dst_v, rows, gsem, ssem, K2)

    plsc.subcore_barrier()
    pltpu.sync_copy(shared.at[pl.ds(base, RPT)],
                    out_hbm.at[c, pl.ds(base, RPT)])

  return seg


def _tc1_body(x_ref, a0_ref, a1_ref, d_ref,
              w1s_ref, w1nt_ref, w1nb_ref, b1_ref, w2n_ref, w2s_ref,
              p_ref, hs_ref):
  inv = 1.0 / jnp.maximum(d_ref[:, :1], 1.0)
  h = (jnp.dot(x_ref[...], w1s_ref[...], preferred_element_type=jnp.float32)
       + jnp.dot(a0_ref[...] * inv, w1nt_ref[...],
                 preferred_element_type=jnp.float32)
       + jnp.dot(a1_ref[:, :D_IN - DC] * inv, w1nb_ref[...],
                 preferred_element_type=jnp.float32)
       + b1_ref[...])
  h = jnp.maximum(h, 0.0)
  p_ref[...] = jnp.dot(h, w2n_ref[...], preferred_element_type=jnp.float32)
  hs_ref[...] = jnp.dot(h, w2s_ref[...], preferred_element_type=jnp.float32)


def _tc2_body(hs_ref, a2A_ref, a2B_ref, d_ref, b2_ref, o_ref):
  inv = 1.0 / jnp.maximum(d_ref[:, :1], 1.0)
  o_ref[...] = (hs_ref[...]
                + (a2A_ref[:, :NCLS] + a2B_ref[:, :NCLS]) * inv
                + b2_ref[...])


_R = 1000


def kernel(x, edge_index, W1_self, W1_neigh, b1, W2_self, W2_neigh, b2):
  f32 = jnp.float32
  src = edge_index[0].astype(jnp.int32)
  dst = edge_index[1].astype(jnp.int32)
  pad = EPAD - E
  src_p = jnp.concatenate([src, jnp.zeros((pad,), jnp.int32)])
  dst_p = jnp.concatenate([dst, jnp.full((pad,), N, jnp.int32)])

  src_t = src_p.reshape(NS, K1, CH)
  src4 = jnp.stack([src_t, src_t + N])
  dst_t = dst_p.reshape(NS, K1, CH)
  src3 = src_p.reshape(NW, K2, CH)
  dst3 = dst_p.reshape(NW, K2, CH)

  t0 = x[:, :DC]
  t1 = jnp.concatenate(
      [x[:, DC:], jnp.ones((N, 1), f32),
       jnp.zeros((N, DC - (D_IN - DC) - 1), f32)], axis=1)
  table = jnp.concatenate([t0, t1], axis=0)

  agg1 = _seg_l1()(table, src4, dst_t)
  a0 = agg1[0, :N]
  a1 = agg1[1, :N]
  dcol = agg1[1, :N, DEGC:DEGC + 8]

  w1sT = W1_self.T
  w1nT_top = W1_neigh.T[:DC]
  w1nT_bot = W1_neigh.T[DC:]
  w2nT = jnp.pad(W2_neigh.T, ((0, 0), (0, DW2 - NCLS)))
  w2sT = W2_self.T
  b1r = b1.reshape(1, D_HID)
  b2r = b2.reshape(1, NCLS)

  grid = (N // _R,)
  row_spec = lambda w: pl.BlockSpec((_R, w), lambda i: (i, 0))
  full_spec = lambda a, b: pl.BlockSpec((a, b), lambda i: (0, 0))

  p, hs = pl.pallas_call(
      _tc1_body,
      grid=grid,
      in_specs=[
          row_spec(D_IN), row_spec(DC), row_spec(DC), row_spec(8),
          full_spec(D_IN, D_HID), full_spec(DC, D_HID),
          full_spec(D_IN - DC, D_HID), full_spec(1, D_HID),
          full_spec(D_HID, DW2), full_spec(D_HID, NCLS),
      ],
      out_specs=[row_spec(DW2), row_spec(NCLS)],
      out_shape=[jax.ShapeDtypeStruct((N, DW2), f32),
                 jax.ShapeDtypeStruct((N, NCLS), f32)],
  )(x, a0, a1, dcol, w1sT, w1nT_top, w1nT_bot, b1r, w2nT, w2sT)

  agg2 = _seg_l2()(p, src3, dst3)

  out = pl.pallas_call(
      _tc2_body,
      grid=grid,
      in_specs=[
          row_spec(NCLS), row_spec(DW2), row_spec(DW2), row_spec(8),
          full_spec(1, NCLS),
      ],
      out_specs=row_spec(NCLS),
      out_shape=jax.ShapeDtypeStruct((N, NCLS), f32),
  )(hs, agg2[0, :N], agg2[1, :N], dcol, b2r)

  return out


# speedup vs baseline: 1.0027x; 1.0027x over previous
import functools
import jax
import jax.numpy as jnp
from jax import lax
from jax.experimental import pallas as pl
from jax.experimental.pallas import tpu as pltpu
from jax.experimental.pallas import tpu_sc as plsc

N = 10000
E = 320000
D_IN = 128
D_HID = 256
NCLS = 40

NC, NS, L = 2, 16, 16
NW = NC * NS
CH = 128
NBUF = 4
K1 = 160
K2 = 80
EPAD = NS * K1 * CH
NPAD = 10112
RPT = NPAD // NS
DC = 80
DEGC = 48
DW2 = 48


def _pipeline(table_hbm, shared, src_v, dst_v, rows, gsem, ssem, n_chunks):
  def g_issue(b, j):
    pltpu.async_copy(table_hbm.at[src_v.at[j]], rows[b], gsem[b])

  def g_wait(b):
    pltpu.make_async_copy(table_hbm.at[src_v.at[0]], rows[b], gsem[b]).wait()

  def s_issue(b, j):
    pltpu.async_copy(rows[b], shared.at[dst_v.at[j]], ssem[b], add=True)

  def s_wait(b):
    pltpu.make_async_copy(rows[b], shared.at[dst_v.at[0]], ssem[b]).wait()

  for b in range(NBUF):
    g_issue(b, b)

  def body(jj, _):
    base_j = jj * NBUF
    for b in range(NBUF):
      g_wait(b)
      s_issue(b, base_j + b)
    for b in range(NBUF):
      s_wait(b)
      g_issue(b, base_j + NBUF + b)
    return _
  lax.fori_loop(0, n_chunks // NBUF - 1, body, 0)

  for b in range(NBUF):
    g_wait(b)
    s_issue(b, n_chunks - NBUF + b)
  for b in range(NBUF):
    s_wait(b)


def _zero_shared_slice(rows0, shared, base, D):
  def zrow(r, _):
    def zcol(q, __):
      rows0[r, pl.ds(q * L, L)] = jnp.zeros((L,), jnp.float32)
      return __
    return lax.fori_loop(0, D // L, zcol, _, unroll=True)
  lax.fori_loop(0, CH, zrow, 0)
  for t in range(RPT // CH):
    pltpu.sync_copy(rows0, shared.at[pl.ds(base + t * CH, CH)])
  rem = RPT % CH
  if rem:
    pltpu.sync_copy(rows0.at[pl.ds(0, rem)],
                    shared.at[pl.ds(base + (RPT // CH) * CH, rem)])


def _mesh():
  return plsc.VectorSubcoreMesh(core_axis_name="c", subcore_axis_name="s",
                                num_cores=NC, num_subcores=NS)


@functools.lru_cache(maxsize=None)
def _seg_l1():
  @functools.partial(
      pl.kernel,
      out_type=jax.ShapeDtypeStruct((NC, NPAD, DC), jnp.float32),
      mesh=_mesh(),
      scratch_types=(
          [pltpu.VMEM((K1 // 2, CH), jnp.int32),
           pltpu.VMEM((K1 // 2, CH), jnp.int32)]
          + [pltpu.VMEM((CH, DC), jnp.float32) for _ in range(NBUF)]
          + [pltpu.SemaphoreType.DMA for _ in range(2 * NBUF)]
          + [pltpu.VMEM_SHARED((NPAD, DC), jnp.float32)]
      ),
      compiler_params=pltpu.CompilerParams(use_tc_tiling_on_sc=False),
  )
  def seg(table_hbm, src_hbm, dst_hbm, out_hbm, src_v, dst_v, *rest):
    rows = rest[:NBUF]
    gsem = rest[NBUF:2 * NBUF]
    ssem = rest[2 * NBUF:3 * NBUF]
    shared = rest[3 * NBUF]
    c = lax.axis_index("c")
    s = lax.axis_index("s")

    base = s * RPT
    K1H = K1 // 2
    for h in range(2):
      ig = pltpu.async_copy(src_hbm.at[c, s, pl.ds(h * K1H, K1H)],
                            src_v, gsem[0])
      ig2 = pltpu.async_copy(dst_hbm.at[s, pl.ds(h * K1H, K1H)],
                             dst_v, gsem[1])
      if h == 0:
        _zero_shared_slice(rows[0], shared, base, DC)
      ig.wait()
      ig2.wait()
      if h == 0:
        plsc.subcore_barrier()
      _pipeline(table_hbm, shared, src_v, dst_v, rows, gsem, ssem, K1H)

    plsc.subcore_barrier()
    pltpu.sync_copy(shared.at[pl.ds(base, RPT)],
                    out_hbm.at[c, pl.ds(base, RPT)])

  return seg


@functools.lru_cache(maxsize=None)
def _seg_l2():
  @functools.partial(
      pl.kernel,
      out_type=jax.ShapeDtypeStruct((NC, NPAD, DW2), jnp.float32),
      mesh=_mesh(),
      scratch_types=(
          [pltpu.VMEM((K2, CH), jnp.int32),
           pltpu.VMEM((K2, CH), jnp.int32)]
          + [pltpu.VMEM((CH, DW2), jnp.float32) for _ in range(NBUF)]
          + [pltpu.SemaphoreType.DMA for _ in range(2 * NBUF)]
          + [pltpu.VMEM_SHARED((NPAD, DW2), jnp.float32)]
      ),
      compiler_params=pltpu.CompilerParams(use_tc_tiling_on_sc=False),
  )
  def seg(table_hbm, src_hbm, dst_hbm, out_hbm, src_v, dst_v, *rest):
    rows = rest[:NBUF]
    gsem = rest[NBUF:2 * NBUF]
    ssem = rest[2 * NBUF:3 * NBUF]
    shared = rest[3 * NBUF]
    c = lax.axis_index("c")
    s = lax.axis_index("s")
    wid = s * NC + c

    ig = pltpu.async_copy(src_hbm.at[wid], src_v, gsem[0])
    ig2 = pltpu.async_copy(dst_hbm.at[wid], dst_v, gsem[1])
    base = s * RPT
    _zero_shared_slice(rows[0], shared, base, DW2)
    ig.wait()
    ig2.wait()
    plsc.subcore_barrier()

    _pipeline(table_hbm, shared, src_v, dst_v, rows, gsem, ssem, K2)

    plsc.subcore_barrier()
    pltpu.sync_copy(shared.at[pl.ds(base, RPT)],
                    out_hbm.at[c, pl.ds(base, RPT)])

  return seg


def _tc1_body(x_ref, a0_ref, a1_ref, d_ref,
              w1s_ref, w1nt_ref, w1nb_ref, b1_ref, w2n_ref, w2s_ref,
              p_ref, hs_ref):
  inv = 1.0 / jnp.maximum(d_ref[:, :1], 1.0)
  h = (jnp.dot(x_ref[...], w1s_ref[...], preferred_element_type=jnp.float32)
       + jnp.dot(a0_ref[...] * inv, w1nt_ref[...],
                 preferred_element_type=jnp.float32)
       + jnp.dot(a1_ref[:, :D_IN - DC] * inv, w1nb_ref[...],
                 preferred_element_type=jnp.float32)
       + b1_ref[...])
  h = jnp.maximum(h, 0.0)
  p_ref[...] = jnp.dot(h, w2n_ref[...], preferred_element_type=jnp.float32)
  hs_ref[...] = jnp.dot(h, w2s_ref[...], preferred_element_type=jnp.float32)


def _tc2_body(hs_ref, a2A_ref, a2B_ref, d_ref, b2_ref, o_ref):
  inv = 1.0 / jnp.maximum(d_ref[:, :1], 1.0)
  o_ref[...] = (hs_ref[...]
                + (a2A_ref[:, :NCLS] + a2B_ref[:, :NCLS]) * inv
                + b2_ref[...])


_R = 1000


def kernel(x, edge_index, W1_self, W1_neigh, b1, W2_self, W2_neigh, b2):
  f32 = jnp.float32
  src = edge_index[0].astype(jnp.int32)
  dst = edge_index[1].astype(jnp.int32)
  pad = EPAD - E
  src_p = jnp.concatenate([src, jnp.zeros((pad,), jnp.int32)])
  dst_pad = N + (jnp.arange(pad, dtype=jnp.int32) % (NPAD - N))
  dst_p = jnp.concatenate([dst, dst_pad])

  src_t = src_p.reshape(NS, K1, CH)
  src4 = jnp.stack([src_t, src_t + N])
  dst_t = dst_p.reshape(NS, K1, CH)
  src3 = src_p.reshape(NW, K2, CH)
  dst3 = dst_p.reshape(NW, K2, CH)

  t0 = x[:, :DC]
  t1 = jnp.concatenate(
      [x[:, DC:], jnp.ones((N, 1), f32),
       jnp.zeros((N, DC - (D_IN - DC) - 1), f32)], axis=1)
  table = jnp.concatenate([t0, t1], axis=0)

  agg1 = _seg_l1()(table, src4, dst_t)
  a0 = agg1[0, :N]
  a1 = agg1[1, :N]
  dcol = agg1[1, :N, DEGC:DEGC + 8]

  w1sT = W1_self.T
  w1nT_top = W1_neigh.T[:DC]
  w1nT_bot = W1_neigh.T[DC:]
  w2nT = jnp.pad(W2_neigh.T, ((0, 0), (0, DW2 - NCLS)))
  w2sT = W2_self.T
  b1r = b1.reshape(1, D_HID)
  b2r = b2.reshape(1, NCLS)

  grid = (N // _R,)
  row_spec = lambda w: pl.BlockSpec((_R, w), lambda i: (i, 0))
  full_spec = lambda a, b: pl.BlockSpec((a, b), lambda i: (0, 0))

  p, hs = pl.pallas_call(
      _tc1_body,
      grid=grid,
      in_specs=[
          row_spec(D_IN), row_spec(DC), row_spec(DC), row_spec(8),
          full_spec(D_IN, D_HID), full_spec(DC, D_HID),
          full_spec(D_IN - DC, D_HID), full_spec(1, D_HID),
          full_spec(D_HID, DW2), full_spec(D_HID, NCLS),
      ],
      out_specs=[row_spec(DW2), row_spec(NCLS)],
      out_shape=[jax.ShapeDtypeStruct((N, DW2), f32),
                 jax.ShapeDtypeStruct((N, NCLS), f32)],
  )(x, a0, a1, dcol, w1sT, w1nT_top, w1nT_bot, b1r, w2nT, w2sT)

  agg2 = _seg_l2()(p, src3, dst3)

  out = pl.pallas_call(
      _tc2_body,
      grid=grid,
      in_specs=[
          row_spec(NCLS), row_spec(DW2), row_spec(DW2), row_spec(8),
          full_spec(1, NCLS),
      ],
      out_specs=row_spec(NCLS),
      out_shape=jax.ShapeDtypeStruct((N, NCLS), f32),
  )(hs, agg2[0, :N], agg2[1, :N], dcol, b2r)

  return out


# speedup vs baseline: 1.7907x; 1.7858x over previous
import functools
import jax
import jax.numpy as jnp
from jax import lax
from jax.experimental import pallas as pl
from jax.experimental.pallas import tpu as pltpu
from jax.experimental.pallas import tpu_sc as plsc

N = 10000
E = 320000
D_IN = 128
D_HID = 256
NCLS = 40

NC, NS, L = 2, 16, 16
NW = NC * NS
CH = 128
NBUF = 4
K1 = 160
K2 = 80
EPAD = NS * K1 * CH
NPAD = 10112
RPT = NPAD // NS
DC = 80
DEGC = 48
DW2 = 48


def _pipeline(table_hbm, shared, src_v, dst_v, rows, gsem, ssem, n_chunks):
  NBUF = len(rows)

  def g_issue(b, j):
    pltpu.async_copy(table_hbm.at[src_v.at[j]], rows[b], gsem[b])

  def g_wait(b):
    pltpu.make_async_copy(table_hbm.at[src_v.at[0]], rows[b], gsem[b]).wait()

  def s_issue(b, j):
    pltpu.async_copy(rows[b], shared.at[dst_v.at[j]], ssem[b], add=True)

  def s_wait(b):
    pltpu.make_async_copy(rows[b], shared.at[dst_v.at[0]], ssem[b]).wait()

  for b in range(NBUF):
    g_issue(b, b)

  def body(jj, _):
    base_j = jj * NBUF
    for b in range(NBUF):
      g_wait(b)
      s_issue(b, base_j + b)
    for b in range(NBUF):
      s_wait(b)
      g_issue(b, base_j + NBUF + b)
    return _
  lax.fori_loop(0, n_chunks // NBUF - 1, body, 0)

  for b in range(NBUF):
    g_wait(b)
    s_issue(b, n_chunks - NBUF + b)
  for b in range(NBUF):
    s_wait(b)


def _zero_shared_slice(rows0, shared, base, D):
  def zrow(r, _):
    def zcol(q, __):
      rows0[r, pl.ds(q * L, L)] = jnp.zeros((L,), jnp.float32)
      return __
    return lax.fori_loop(0, D // L, zcol, _, unroll=True)
  lax.fori_loop(0, CH, zrow, 0)
  for t in range(RPT // CH):
    pltpu.sync_copy(rows0, shared.at[pl.ds(base + t * CH, CH)])
  rem = RPT % CH
  if rem:
    pltpu.sync_copy(rows0.at[pl.ds(0, rem)],
                    shared.at[pl.ds(base + (RPT // CH) * CH, rem)])


def _mesh():
  return plsc.VectorSubcoreMesh(core_axis_name="c", subcore_axis_name="s",
                                num_cores=NC, num_subcores=NS)


_RLAST = N - (NS - 1) * RPT


def _load_table(table_hbm, table_sh, hbm_base, s):
  @pl.when(s < NS - 1)
  def _():
    pltpu.sync_copy(table_hbm.at[pl.ds(hbm_base + s * RPT, RPT)],
                    table_sh.at[pl.ds(s * RPT, RPT)])
  @pl.when(s == NS - 1)
  def _():
    pltpu.sync_copy(table_hbm.at[pl.ds(hbm_base + (NS - 1) * RPT, _RLAST)],
                    table_sh.at[pl.ds((NS - 1) * RPT, _RLAST)])


NB1 = 2
ST1 = 8


@functools.lru_cache(maxsize=None)
def _seg_l1():
  @functools.partial(
      pl.kernel,
      out_type=jax.ShapeDtypeStruct((NC, NPAD, DC), jnp.float32),
      mesh=_mesh(),
      scratch_types=(
          [pltpu.VMEM((K1 // ST1, CH), jnp.int32),
           pltpu.VMEM((K1 // ST1, CH), jnp.int32)]
          + [pltpu.VMEM((CH, DC), jnp.float32) for _ in range(NB1)]
          + [pltpu.SemaphoreType.DMA for _ in range(2 * NB1)]
          + [pltpu.VMEM_SHARED((NPAD, DC), jnp.float32),
             pltpu.VMEM_SHARED((N, DC), jnp.float32)]
      ),
      compiler_params=pltpu.CompilerParams(use_tc_tiling_on_sc=False),
  )
  def seg(table_hbm, src_hbm, dst_hbm, out_hbm, src_v, dst_v, *rest):
    rows = rest[:NB1]
    gsem = rest[NB1:2 * NB1]
    ssem = rest[2 * NB1:3 * NB1]
    shared = rest[3 * NB1]
    table_sh = rest[3 * NB1 + 1]
    c = lax.axis_index("c")
    s = lax.axis_index("s")

    base = s * RPT
    K1S = K1 // ST1
    for h in range(ST1):
      ig = pltpu.async_copy(src_hbm.at[s, pl.ds(h * K1S, K1S)],
                            src_v, gsem[0])
      ig2 = pltpu.async_copy(dst_hbm.at[s, pl.ds(h * K1S, K1S)],
                             dst_v, gsem[1])
      if h == 0:
        _load_table(table_hbm, table_sh, c * N, s)
        _zero_shared_slice(rows[0], shared, base, DC)
      ig.wait()
      ig2.wait()
      if h == 0:
        plsc.subcore_barrier()
      _pipeline(table_sh, shared, src_v, dst_v, rows, gsem, ssem, K1S)

    plsc.subcore_barrier()
    pltpu.sync_copy(shared.at[pl.ds(base, RPT)],
                    out_hbm.at[c, pl.ds(base, RPT)])

  return seg


@functools.lru_cache(maxsize=None)
def _seg_l2():
  @functools.partial(
      pl.kernel,
      out_type=jax.ShapeDtypeStruct((NC, NPAD, DW2), jnp.float32),
      mesh=_mesh(),
      scratch_types=(
          [pltpu.VMEM((K2, CH), jnp.int32),
           pltpu.VMEM((K2, CH), jnp.int32)]
          + [pltpu.VMEM((CH, DW2), jnp.float32) for _ in range(NBUF)]
          + [pltpu.SemaphoreType.DMA for _ in range(2 * NBUF)]
          + [pltpu.VMEM_SHARED((NPAD, DW2), jnp.float32),
             pltpu.VMEM_SHARED((N, DW2), jnp.float32)]
      ),
      compiler_params=pltpu.CompilerParams(use_tc_tiling_on_sc=False),
  )
  def seg(table_hbm, src_hbm, dst_hbm, out_hbm, src_v, dst_v, *rest):
    rows = rest[:NBUF]
    gsem = rest[NBUF:2 * NBUF]
    ssem = rest[2 * NBUF:3 * NBUF]
    shared = rest[3 * NBUF]
    table_sh = rest[3 * NBUF + 1]
    c = lax.axis_index("c")
    s = lax.axis_index("s")
    wid = s * NC + c

    ig = pltpu.async_copy(src_hbm.at[wid], src_v, gsem[0])
    ig2 = pltpu.async_copy(dst_hbm.at[wid], dst_v, gsem[1])
    base = s * RPT
    _load_table(table_hbm, table_sh, 0, s)
    _zero_shared_slice(rows[0], shared, base, DW2)
    ig.wait()
    ig2.wait()
    plsc.subcore_barrier()

    _pipeline(table_sh, shared, src_v, dst_v, rows, gsem, ssem, K2)

    plsc.subcore_barrier()
    pltpu.sync_copy(shared.at[pl.ds(base, RPT)],
                    out_hbm.at[c, pl.ds(base, RPT)])

  return seg


def _tc1_body(x_ref, a0_ref, a1_ref, d_ref,
              w1s_ref, w1nt_ref, w1nb_ref, b1_ref, w2n_ref, w2s_ref,
              p_ref, hs_ref):
  inv = 1.0 / jnp.maximum(d_ref[:, :1], 1.0)
  h = (jnp.dot(x_ref[...], w1s_ref[...], preferred_element_type=jnp.float32)
       + jnp.dot(a0_ref[...] * inv, w1nt_ref[...],
                 preferred_element_type=jnp.float32)
       + jnp.dot(a1_ref[:, :D_IN - DC] * inv, w1nb_ref[...],
                 preferred_element_type=jnp.float32)
       + b1_ref[...])
  h = jnp.maximum(h, 0.0)
  p_ref[...] = jnp.dot(h, w2n_ref[...], preferred_element_type=jnp.float32)
  hs_ref[...] = jnp.dot(h, w2s_ref[...], preferred_element_type=jnp.float32)


def _tc2_body(hs_ref, a2A_ref, a2B_ref, d_ref, b2_ref, o_ref):
  inv = 1.0 / jnp.maximum(d_ref[:, :1], 1.0)
  o_ref[...] = (hs_ref[...]
                + (a2A_ref[:, :NCLS] + a2B_ref[:, :NCLS]) * inv
                + b2_ref[...])


_R = 1000


def kernel(x, edge_index, W1_self, W1_neigh, b1, W2_self, W2_neigh, b2):
  f32 = jnp.float32
  src = edge_index[0].astype(jnp.int32)
  dst = edge_index[1].astype(jnp.int32)
  pad = EPAD - E
  src_p = jnp.concatenate([src, jnp.zeros((pad,), jnp.int32)])
  dst_pad = N + (jnp.arange(pad, dtype=jnp.int32) % (NPAD - N))
  dst_p = jnp.concatenate([dst, dst_pad])

  src_t = src_p.reshape(NS, K1, CH)
  dst_t = dst_p.reshape(NS, K1, CH)
  src3 = src_p.reshape(NW, K2, CH)
  dst3 = dst_p.reshape(NW, K2, CH)

  t0 = x[:, :DC]
  t1 = jnp.concatenate(
      [x[:, DC:], jnp.ones((N, 1), f32),
       jnp.zeros((N, DC - (D_IN - DC) - 1), f32)], axis=1)
  table = jnp.concatenate([t0, t1], axis=0)

  agg1 = _seg_l1()(table, src_t, dst_t)
  a0 = agg1[0, :N]
  a1 = agg1[1, :N]
  dcol = agg1[1, :N, DEGC:DEGC + 8]

  w1sT = W1_self.T
  w1nT_top = W1_neigh.T[:DC]
  w1nT_bot = W1_neigh.T[DC:]
  w2nT = jnp.pad(W2_neigh.T, ((0, 0), (0, DW2 - NCLS)))
  w2sT = W2_self.T
  b1r = b1.reshape(1, D_HID)
  b2r = b2.reshape(1, NCLS)

  grid = (N // _R,)
  row_spec = lambda w: pl.BlockSpec((_R, w), lambda i: (i, 0))
  full_spec = lambda a, b: pl.BlockSpec((a, b), lambda i: (0, 0))

  p, hs = pl.pallas_call(
      _tc1_body,
      grid=grid,
      in_specs=[
          row_spec(D_IN), row_spec(DC), row_spec(DC), row_spec(8),
          full_spec(D_IN, D_HID), full_spec(DC, D_HID),
          full_spec(D_IN - DC, D_HID), full_spec(1, D_HID),
          full_spec(D_HID, DW2), full_spec(D_HID, NCLS),
      ],
      out_specs=[row_spec(DW2), row_spec(NCLS)],
      out_shape=[jax.ShapeDtypeStruct((N, DW2), f32),
                 jax.ShapeDtypeStruct((N, NCLS), f32)],
  )(x, a0, a1, dcol, w1sT, w1nT_top, w1nT_bot, b1r, w2nT, w2sT)

  agg2 = _seg_l2()(p, src3, dst3)

  out = pl.pallas_call(
      _tc2_body,
      grid=grid,
      in_specs=[
          row_spec(NCLS), row_spec(DW2), row_spec(DW2), row_spec(8),
          full_spec(1, NCLS),
      ],
      out_specs=row_spec(NCLS),
      out_shape=jax.ShapeDtypeStruct((N, NCLS), f32),
  )(hs, agg2[0, :N], agg2[1, :N], dcol, b2r)

  return out


# speedup vs baseline: 1.8774x; 1.0484x over previous
import functools
import jax
import jax.numpy as jnp
from jax import lax
from jax.experimental import pallas as pl
from jax.experimental.pallas import tpu as pltpu
from jax.experimental.pallas import tpu_sc as plsc

N = 10000
E = 320000
D_IN = 128
D_HID = 256
NCLS = 40

NC, NS, L = 2, 16, 16
NW = NC * NS
CH = 128
NBUF = 4
K1 = 160
K2 = 80
EPAD = NS * K1 * CH
NPAD = 10112
RPT = NPAD // NS
DC = 80
DEGC = 48
DW2 = 48


def _pipeline(table_hbm, shared, src_v, dst_v, rows, gsem, ssem, n_chunks):
  NBUF = len(rows)

  def g_issue(b, j):
    pltpu.async_copy(table_hbm.at[src_v.at[j]], rows[b], gsem[b])

  def g_wait(b):
    pltpu.make_async_copy(table_hbm.at[src_v.at[0]], rows[b], gsem[b]).wait()

  def s_issue(b, j):
    pltpu.async_copy(rows[b], shared.at[dst_v.at[j]], ssem[b], add=True)

  def s_wait(b):
    pltpu.make_async_copy(rows[b], shared.at[dst_v.at[0]], ssem[b]).wait()

  for b in range(NBUF):
    g_issue(b, b)

  def body(jj, _):
    base_j = jj * NBUF
    for b in range(NBUF):
      g_wait(b)
      s_issue(b, base_j + b)
    for b in range(NBUF):
      s_wait(b)
      g_issue(b, base_j + NBUF + b)
    return _
  lax.fori_loop(0, n_chunks // NBUF - 1, body, 0)

  for b in range(NBUF):
    g_wait(b)
    s_issue(b, n_chunks - NBUF + b)
  for b in range(NBUF):
    s_wait(b)


def _zero_shared_slice(rows0, shared, base, D):
  def zrow(r, _):
    def zcol(q, __):
      rows0[r, pl.ds(q * L, L)] = jnp.zeros((L,), jnp.float32)
      return __
    return lax.fori_loop(0, D // L, zcol, _, unroll=True)
  lax.fori_loop(0, CH, zrow, 0)
  for t in range(RPT // CH):
    pltpu.sync_copy(rows0, shared.at[pl.ds(base + t * CH, CH)])
  rem = RPT % CH
  if rem:
    pltpu.sync_copy(rows0.at[pl.ds(0, rem)],
                    shared.at[pl.ds(base + (RPT // CH) * CH, rem)])


def _mesh():
  return plsc.VectorSubcoreMesh(core_axis_name="c", subcore_axis_name="s",
                                num_cores=NC, num_subcores=NS)


_RLAST = N - (NS - 1) * RPT


def _load_table(table_hbm, table_sh, hbm_base, s):
  @pl.when(s < NS - 1)
  def _():
    pltpu.sync_copy(table_hbm.at[pl.ds(hbm_base + s * RPT, RPT)],
                    table_sh.at[pl.ds(s * RPT, RPT)])
  @pl.when(s == NS - 1)
  def _():
    pltpu.sync_copy(table_hbm.at[pl.ds(hbm_base + (NS - 1) * RPT, _RLAST)],
                    table_sh.at[pl.ds((NS - 1) * RPT, _RLAST)])


NB1 = 2
ST1 = 8


@functools.lru_cache(maxsize=None)
def _seg_l1():
  @functools.partial(
      pl.kernel,
      out_type=jax.ShapeDtypeStruct((NC, NPAD, DC), jnp.float32),
      mesh=_mesh(),
      scratch_types=(
          [pltpu.VMEM((K1 // ST1, CH), jnp.int32),
           pltpu.VMEM((K1 // ST1, CH), jnp.int32)]
          + [pltpu.VMEM((CH, DC), jnp.float32) for _ in range(NB1)]
          + [pltpu.SemaphoreType.DMA for _ in range(2 * NB1)]
          + [pltpu.VMEM_SHARED((NPAD, DC), jnp.float32),
             pltpu.VMEM_SHARED((N, DC), jnp.float32)]
      ),
      compiler_params=pltpu.CompilerParams(use_tc_tiling_on_sc=False),
  )
  def seg(table_hbm, src_hbm, dst_hbm, out_hbm, src_v, dst_v, *rest):
    rows = rest[:NB1]
    gsem = rest[NB1:2 * NB1]
    ssem = rest[2 * NB1:3 * NB1]
    shared = rest[3 * NB1]
    table_sh = rest[3 * NB1 + 1]
    c = lax.axis_index("c")
    s = lax.axis_index("s")

    base = s * RPT
    K1S = K1 // ST1
    for h in range(ST1):
      ig = pltpu.async_copy(src_hbm.at[s, pl.ds(h * K1S, K1S)],
                            src_v, gsem[0])
      ig2 = pltpu.async_copy(dst_hbm.at[s, pl.ds(h * K1S, K1S)],
                             dst_v, gsem[1])
      if h == 0:
        _load_table(table_hbm, table_sh, c * N, s)
        _zero_shared_slice(rows[0], shared, base, DC)
      ig.wait()
      ig2.wait()
      if h == 0:
        plsc.subcore_barrier()
      _pipeline(table_sh, shared, src_v, dst_v, rows, gsem, ssem, K1S)

    plsc.subcore_barrier()
    pltpu.sync_copy(shared.at[pl.ds(base, RPT)],
                    out_hbm.at[c, pl.ds(base, RPT)])

  return seg


@functools.lru_cache(maxsize=None)
def _seg_l2():
  @functools.partial(
      pl.kernel,
      out_type=jax.ShapeDtypeStruct((NC, NPAD, DW2), jnp.float32),
      mesh=_mesh(),
      scratch_types=(
          [pltpu.VMEM((K2, CH), jnp.int32),
           pltpu.VMEM((K2, CH), jnp.int32)]
          + [pltpu.VMEM((CH, DW2), jnp.float32) for _ in range(NBUF)]
          + [pltpu.SemaphoreType.DMA for _ in range(2 * NBUF)]
          + [pltpu.VMEM_SHARED((NPAD, DW2), jnp.float32),
             pltpu.VMEM_SHARED((N, DW2), jnp.float32)]
      ),
      compiler_params=pltpu.CompilerParams(use_tc_tiling_on_sc=False),
  )
  def seg(table_hbm, src_hbm, dst_hbm, out_hbm, src_v, dst_v, *rest):
    rows = rest[:NBUF]
    gsem = rest[NBUF:2 * NBUF]
    ssem = rest[2 * NBUF:3 * NBUF]
    shared = rest[3 * NBUF]
    table_sh = rest[3 * NBUF + 1]
    c = lax.axis_index("c")
    s = lax.axis_index("s")
    wid = s * NC + c

    ig = pltpu.async_copy(src_hbm.at[wid], src_v, gsem[0])
    ig2 = pltpu.async_copy(dst_hbm.at[wid], dst_v, gsem[1])
    base = s * RPT
    _load_table(table_hbm, table_sh, 0, s)
    _zero_shared_slice(rows[0], shared, base, DW2)
    ig.wait()
    ig2.wait()
    plsc.subcore_barrier()

    _pipeline(table_sh, shared, src_v, dst_v, rows, gsem, ssem, K2)

    plsc.subcore_barrier()
    pltpu.sync_copy(shared.at[pl.ds(base, RPT)],
                    out_hbm.at[c, pl.ds(base, RPT)])

  return seg


def _tc1_body(x_ref, a0_ref, a1_ref,
              w1s_ref, w1nt_ref, w1nb_ref, b1_ref, w2n_ref, w2s_ref,
              p_ref, hs_ref):
  inv = 1.0 / jnp.maximum(a1_ref[0, :, DEGC:DEGC + 1], 1.0)
  h = (jnp.dot(x_ref[...], w1s_ref[...], preferred_element_type=jnp.float32)
       + jnp.dot(a0_ref[0] * inv, w1nt_ref[...],
                 preferred_element_type=jnp.float32)
       + jnp.dot(a1_ref[0, :, :D_IN - DC] * inv, w1nb_ref[...],
                 preferred_element_type=jnp.float32)
       + b1_ref[...])
  h = jnp.maximum(h, 0.0)
  p_ref[...] = jnp.dot(h, w2n_ref[...], preferred_element_type=jnp.float32)
  hs_ref[...] = jnp.dot(h, w2s_ref[...], preferred_element_type=jnp.float32)


def _tc2_body(hs_ref, a2_ref, a1_ref, b2_ref, o_ref):
  inv = 1.0 / jnp.maximum(a1_ref[0, :, DEGC:DEGC + 1], 1.0)
  o_ref[...] = (hs_ref[...]
                + (a2_ref[0, :, :NCLS] + a2_ref[1, :, :NCLS]) * inv
                + b2_ref[...])


_R = 1000


def kernel(x, edge_index, W1_self, W1_neigh, b1, W2_self, W2_neigh, b2):
  f32 = jnp.float32
  src = edge_index[0].astype(jnp.int32)
  dst = edge_index[1].astype(jnp.int32)
  pad = EPAD - E
  src_p = jnp.concatenate([src, jnp.zeros((pad,), jnp.int32)])
  dst_pad = N + (jnp.arange(pad, dtype=jnp.int32) % (NPAD - N))
  dst_p = jnp.concatenate([dst, dst_pad])

  src_t = src_p.reshape(NS, K1, CH)
  dst_t = dst_p.reshape(NS, K1, CH)
  src3 = src_p.reshape(NW, K2, CH)
  dst3 = dst_p.reshape(NW, K2, CH)

  t0 = x[:, :DC]
  t1 = jnp.concatenate(
      [x[:, DC:], jnp.ones((N, 1), f32),
       jnp.zeros((N, DC - (D_IN - DC) - 1), f32)], axis=1)
  table = jnp.concatenate([t0, t1], axis=0)

  agg1 = _seg_l1()(table, src_t, dst_t)

  w1sT = W1_self.T
  w1nT_top = W1_neigh.T[:DC]
  w1nT_bot = W1_neigh.T[DC:]
  w2nT = jnp.pad(W2_neigh.T, ((0, 0), (0, DW2 - NCLS)))
  w2sT = W2_self.T
  b1r = b1.reshape(1, D_HID)
  b2r = b2.reshape(1, NCLS)

  grid = (N // _R,)
  row_spec = lambda w: pl.BlockSpec((_R, w), lambda i: (i, 0))
  full_spec = lambda a, b: pl.BlockSpec((a, b), lambda i: (0, 0))
  agg_spec = lambda h, w, j: pl.BlockSpec((1, _R, w), lambda i: (h, i, j))

  p, hs = pl.pallas_call(
      _tc1_body,
      grid=grid,
      in_specs=[
          row_spec(D_IN), agg_spec(0, DC, 0), agg_spec(1, DC, 0),
          full_spec(D_IN, D_HID), full_spec(DC, D_HID),
          full_spec(D_IN - DC, D_HID), full_spec(1, D_HID),
          full_spec(D_HID, DW2), full_spec(D_HID, NCLS),
      ],
      out_specs=[row_spec(DW2), row_spec(NCLS)],
      out_shape=[jax.ShapeDtypeStruct((N, DW2), f32),
                 jax.ShapeDtypeStruct((N, NCLS), f32)],
  )(x, agg1, agg1, w1sT, w1nT_top, w1nT_bot, b1r, w2nT, w2sT)

  agg2 = _seg_l2()(p, src3, dst3)

  out = pl.pallas_call(
      _tc2_body,
      grid=grid,
      in_specs=[
          row_spec(NCLS), pl.BlockSpec((2, _R, DW2), lambda i: (0, i, 0)),
          agg_spec(1, DC, 0), full_spec(1, NCLS),
      ],
      out_specs=row_spec(NCLS),
      out_shape=jax.ShapeDtypeStruct((N, NCLS), f32),
  )(hs, agg2, agg1, b2r)

  return out


# speedup vs baseline: 1.9620x; 1.0451x over previous
import functools
import jax
import jax.numpy as jnp
from jax import lax
from jax.experimental import pallas as pl
from jax.experimental.pallas import tpu as pltpu
from jax.experimental.pallas import tpu_sc as plsc

N = 10000
E = 320000
D_IN = 128
D_HID = 256
NCLS = 40

NC, NS, L = 2, 16, 16
NW = NC * NS
CH = 128
CH1 = 64
NBUF = 4
K1 = 320
K2 = 80
EPAD = NS * K1 * CH1
NPAD = 10112
RPT = NPAD // NS
DC = 80
DEGC = 48
DW2 = 48


def _pipeline(table_hbm, shared, src_v, dst_v, rows, gsem, ssem, n_chunks):
  NBUF = len(rows)

  def g_issue(b, j):
    pltpu.async_copy(table_hbm.at[src_v.at[j]], rows[b], gsem[b])

  def g_wait(b):
    pltpu.make_async_copy(table_hbm.at[src_v.at[0]], rows[b], gsem[b]).wait()

  def s_issue(b, j):
    pltpu.async_copy(rows[b], shared.at[dst_v.at[j]], ssem[b], add=True)

  def s_wait(b):
    pltpu.make_async_copy(rows[b], shared.at[dst_v.at[0]], ssem[b]).wait()

  for b in range(NBUF):
    g_issue(b, b)

  def body(jj, _):
    base_j = jj * NBUF
    for b in range(NBUF):
      g_wait(b)
      s_issue(b, base_j + b)
    for b in range(NBUF):
      s_wait(b)
      g_issue(b, base_j + NBUF + b)
    return _
  lax.fori_loop(0, n_chunks // NBUF - 1, body, 0)

  for b in range(NBUF):
    g_wait(b)
    s_issue(b, n_chunks - NBUF + b)
  for b in range(NBUF):
    s_wait(b)


def _zero_shared_slice(rows0, shared, base, D):
  nr = rows0.shape[0]
  def zrow(r, _):
    def zcol(q, __):
      rows0[r, pl.ds(q * L, L)] = jnp.zeros((L,), jnp.float32)
      return __
    return lax.fori_loop(0, D // L, zcol, _, unroll=True)
  lax.fori_loop(0, nr, zrow, 0)
  for t in range(RPT // nr):
    pltpu.sync_copy(rows0, shared.at[pl.ds(base + t * nr, nr)])
  rem = RPT % nr
  if rem:
    pltpu.sync_copy(rows0.at[pl.ds(0, rem)],
                    shared.at[pl.ds(base + (RPT // nr) * nr, rem)])


def _mesh():
  return plsc.VectorSubcoreMesh(core_axis_name="c", subcore_axis_name="s",
                                num_cores=NC, num_subcores=NS)


_RLAST = N - (NS - 1) * RPT


def _load_table(table_hbm, table_sh, hbm_base, s):
  @pl.when(s < NS - 1)
  def _():
    pltpu.sync_copy(table_hbm.at[pl.ds(hbm_base + s * RPT, RPT)],
                    table_sh.at[pl.ds(s * RPT, RPT)])
  @pl.when(s == NS - 1)
  def _():
    pltpu.sync_copy(table_hbm.at[pl.ds(hbm_base + (NS - 1) * RPT, _RLAST)],
                    table_sh.at[pl.ds((NS - 1) * RPT, _RLAST)])


NB1 = 4
ST1 = 8


@functools.lru_cache(maxsize=None)
def _seg_l1():
  @functools.partial(
      pl.kernel,
      out_type=jax.ShapeDtypeStruct((NC, NPAD, DC), jnp.float32),
      mesh=_mesh(),
      scratch_types=(
          [pltpu.VMEM((K1 // ST1, CH1), jnp.int32),
           pltpu.VMEM((K1 // ST1, CH1), jnp.int32)]
          + [pltpu.VMEM((CH1, DC), jnp.float32) for _ in range(NB1)]
          + [pltpu.SemaphoreType.DMA for _ in range(2 * NB1)]
          + [pltpu.VMEM_SHARED((NPAD, DC), jnp.float32),
             pltpu.VMEM_SHARED((N, DC), jnp.float32)]
      ),
      compiler_params=pltpu.CompilerParams(use_tc_tiling_on_sc=False),
  )
  def seg(table_hbm, src_hbm, dst_hbm, out_hbm, src_v, dst_v, *rest):
    rows = rest[:NB1]
    gsem = rest[NB1:2 * NB1]
    ssem = rest[2 * NB1:3 * NB1]
    shared = rest[3 * NB1]
    table_sh = rest[3 * NB1 + 1]
    c = lax.axis_index("c")
    s = lax.axis_index("s")

    base = s * RPT
    K1S = K1 // ST1
    for h in range(ST1):
      ig = pltpu.async_copy(src_hbm.at[s, pl.ds(h * K1S, K1S)],
                            src_v, gsem[0])
      ig2 = pltpu.async_copy(dst_hbm.at[s, pl.ds(h * K1S, K1S)],
                             dst_v, gsem[1])
      if h == 0:
        _load_table(table_hbm, table_sh, c * N, s)
        _zero_shared_slice(rows[0], shared, base, DC)
      ig.wait()
      ig2.wait()
      if h == 0:
        plsc.subcore_barrier()
      _pipeline(table_sh, shared, src_v, dst_v, rows, gsem, ssem, K1S)

    plsc.subcore_barrier()
    pltpu.sync_copy(shared.at[pl.ds(base, RPT)],
                    out_hbm.at[c, pl.ds(base, RPT)])

  return seg


@functools.lru_cache(maxsize=None)
def _seg_l2():
  @functools.partial(
      pl.kernel,
      out_type=jax.ShapeDtypeStruct((NC, NPAD, DW2), jnp.float32),
      mesh=_mesh(),
      scratch_types=(
          [pltpu.VMEM((K2, CH), jnp.int32),
           pltpu.VMEM((K2, CH), jnp.int32)]
          + [pltpu.VMEM((CH, DW2), jnp.float32) for _ in range(NBUF)]
          + [pltpu.SemaphoreType.DMA for _ in range(2 * NBUF)]
          + [pltpu.VMEM_SHARED((NPAD, DW2), jnp.float32),
             pltpu.VMEM_SHARED((N, DW2), jnp.float32)]
      ),
      compiler_params=pltpu.CompilerParams(use_tc_tiling_on_sc=False),
  )
  def seg(table_hbm, src_hbm, dst_hbm, out_hbm, src_v, dst_v, *rest):
    rows = rest[:NBUF]
    gsem = rest[NBUF:2 * NBUF]
    ssem = rest[2 * NBUF:3 * NBUF]
    shared = rest[3 * NBUF]
    table_sh = rest[3 * NBUF + 1]
    c = lax.axis_index("c")
    s = lax.axis_index("s")
    wid = s * NC + c

    ig = pltpu.async_copy(src_hbm.at[wid], src_v, gsem[0])
    ig2 = pltpu.async_copy(dst_hbm.at[wid], dst_v, gsem[1])
    base = s * RPT
    _load_table(table_hbm, table_sh, 0, s)
    _zero_shared_slice(rows[0], shared, base, DW2)
    ig.wait()
    ig2.wait()
    plsc.subcore_barrier()

    _pipeline(table_sh, shared, src_v, dst_v, rows, gsem, ssem, K2)

    plsc.subcore_barrier()
    pltpu.sync_copy(shared.at[pl.ds(base, RPT)],
                    out_hbm.at[c, pl.ds(base, RPT)])

  return seg


def _tc1_body(x_ref, a0_ref, a1_ref,
              w1s_ref, w1nt_ref, w1nb_ref, b1_ref, w2n_ref, w2s_ref,
              p_ref, hs_ref):
  inv = 1.0 / jnp.maximum(a1_ref[0, :, DEGC:DEGC + 1], 1.0)
  h = (jnp.dot(x_ref[...], w1s_ref[...], preferred_element_type=jnp.float32)
       + jnp.dot(a0_ref[0] * inv, w1nt_ref[...],
                 preferred_element_type=jnp.float32)
       + jnp.dot(a1_ref[0, :, :D_IN - DC] * inv, w1nb_ref[...],
                 preferred_element_type=jnp.float32)
       + b1_ref[...])
  h = jnp.maximum(h, 0.0)
  p_ref[...] = jnp.dot(h, w2n_ref[...], preferred_element_type=jnp.float32)
  hs_ref[...] = jnp.dot(h, w2s_ref[...], preferred_element_type=jnp.float32)


def _tc2_body(hs_ref, a2_ref, a1_ref, b2_ref, o_ref):
  inv = 1.0 / jnp.maximum(a1_ref[0, :, DEGC:DEGC + 1], 1.0)
  o_ref[...] = (hs_ref[...]
                + (a2_ref[0, :, :NCLS] + a2_ref[1, :, :NCLS]) * inv
                + b2_ref[...])


_R = 1000


def kernel(x, edge_index, W1_self, W1_neigh, b1, W2_self, W2_neigh, b2):
  f32 = jnp.float32
  src = edge_index[0].astype(jnp.int32)
  dst = edge_index[1].astype(jnp.int32)
  pad = EPAD - E
  src_p = jnp.concatenate([src, jnp.zeros((pad,), jnp.int32)])
  dst_pad = N + (jnp.arange(pad, dtype=jnp.int32) % (NPAD - N))
  dst_p = jnp.concatenate([dst, dst_pad])

  src_t = src_p.reshape(NS, K1, CH1)
  dst_t = dst_p.reshape(NS, K1, CH1)
  src3 = src_p.reshape(NW, K2, CH)
  dst3 = dst_p.reshape(NW, K2, CH)

  t0 = x[:, :DC]
  t1 = jnp.concatenate(
      [x[:, DC:], jnp.ones((N, 1), f32),
       jnp.zeros((N, DC - (D_IN - DC) - 1), f32)], axis=1)
  table = jnp.concatenate([t0, t1], axis=0)

  agg1 = _seg_l1()(table, src_t, dst_t)

  w1sT = W1_self.T
  w1nT_top = W1_neigh.T[:DC]
  w1nT_bot = W1_neigh.T[DC:]
  w2nT = jnp.pad(W2_neigh.T, ((0, 0), (0, DW2 - NCLS)))
  w2sT = W2_self.T
  b1r = b1.reshape(1, D_HID)
  b2r = b2.reshape(1, NCLS)

  grid = (N // _R,)
  row_spec = lambda w: pl.BlockSpec((_R, w), lambda i: (i, 0))
  full_spec = lambda a, b: pl.BlockSpec((a, b), lambda i: (0, 0))
  agg_spec = lambda h, w, j: pl.BlockSpec((1, _R, w), lambda i: (h, i, j))

  p, hs = pl.pallas_call(
      _tc1_body,
      grid=grid,
      in_specs=[
          row_spec(D_IN), agg_spec(0, DC, 0), agg_spec(1, DC, 0),
          full_spec(D_IN, D_HID), full_spec(DC, D_HID),
          full_spec(D_IN - DC, D_HID), full_spec(1, D_HID),
          full_spec(D_HID, DW2), full_spec(D_HID, NCLS),
      ],
      out_specs=[row_spec(DW2), row_spec(NCLS)],
      out_shape=[jax.ShapeDtypeStruct((N, DW2), f32),
                 jax.ShapeDtypeStruct((N, NCLS), f32)],
  )(x, agg1, agg1, w1sT, w1nT_top, w1nT_bot, b1r, w2nT, w2sT)

  agg2 = _seg_l2()(p, src3, dst3)

  out = pl.pallas_call(
      _tc2_body,
      grid=grid,
      in_specs=[
          row_spec(NCLS), pl.BlockSpec((2, _R, DW2), lambda i: (0, i, 0)),
          agg_spec(1, DC, 0), full_spec(1, NCLS),
      ],
      out_specs=row_spec(NCLS),
      out_shape=jax.ShapeDtypeStruct((N, NCLS), f32),
  )(hs, agg2, agg1, b2r)

  return out


# speedup vs baseline: 2.0977x; 1.0692x over previous
import functools
import jax
import jax.numpy as jnp
from jax import lax
from jax.experimental import pallas as pl
from jax.experimental.pallas import tpu as pltpu
from jax.experimental.pallas import tpu_sc as plsc

N = 10000
E = 320000
D_IN = 128
D_HID = 256
NCLS = 40

NC, NS, L = 2, 16, 16
NW = NC * NS
CH = 128
CH1 = 64
NBUF = 4
K1 = 320
K2 = 80
EPAD = NS * K1 * CH1
NPAD = 10112
RPT = NPAD // NS
DC = 80
DEGC = 48
DW2 = 48


def _pipeline(table_hbm, shared, src_v, dst_v, rows, gsem, ssem, n_chunks):
  NBUF = len(rows)

  def g_issue(b, j):
    pltpu.async_copy(table_hbm.at[src_v.at[j]], rows[b], gsem[b])

  def g_wait(b):
    pltpu.make_async_copy(table_hbm.at[src_v.at[0]], rows[b], gsem[b]).wait()

  def s_issue(b, j):
    pltpu.async_copy(rows[b], shared.at[dst_v.at[j]], ssem[b], add=True)

  def s_wait(b):
    pltpu.make_async_copy(rows[b], shared.at[dst_v.at[0]], ssem[b]).wait()

  for b in range(NBUF):
    g_issue(b, b)

  def body(jj, _):
    base_j = jj * NBUF
    for b in range(NBUF):
      g_wait(b)
      s_issue(b, base_j + b)
    for b in range(NBUF):
      s_wait(b)
      g_issue(b, base_j + NBUF + b)
    return _
  lax.fori_loop(0, n_chunks // NBUF - 1, body, 0)

  for b in range(NBUF):
    g_wait(b)
    s_issue(b, n_chunks - NBUF + b)
  for b in range(NBUF):
    s_wait(b)


def _zero_shared_slice(rows0, shared, base, D):
  nr = rows0.shape[0]
  def zrow(r, _):
    def zcol(q, __):
      rows0[r, pl.ds(q * L, L)] = jnp.zeros((L,), jnp.float32)
      return __
    return lax.fori_loop(0, D // L, zcol, _, unroll=True)
  lax.fori_loop(0, nr, zrow, 0)
  for t in range(RPT // nr):
    pltpu.sync_copy(rows0, shared.at[pl.ds(base + t * nr, nr)])
  rem = RPT % nr
  if rem:
    pltpu.sync_copy(rows0.at[pl.ds(0, rem)],
                    shared.at[pl.ds(base + (RPT // nr) * nr, rem)])


def _mesh():
  return plsc.VectorSubcoreMesh(core_axis_name="c", subcore_axis_name="s",
                                num_cores=NC, num_subcores=NS)


_RLAST = N - (NS - 1) * RPT


def _load_table(table_hbm, table_sh, hbm_base, s):
  @pl.when(s < NS - 1)
  def _():
    pltpu.sync_copy(table_hbm.at[pl.ds(hbm_base + s * RPT, RPT)],
                    table_sh.at[pl.ds(s * RPT, RPT)])
  @pl.when(s == NS - 1)
  def _():
    pltpu.sync_copy(table_hbm.at[pl.ds(hbm_base + (NS - 1) * RPT, _RLAST)],
                    table_sh.at[pl.ds((NS - 1) * RPT, _RLAST)])


def _build_table_l1(x_hbm, table_sh, ones_buf, c, s):
  def build(nrows):
    base = s * RPT

    @pl.when(c == 0)
    def _():
      pltpu.sync_copy(x_hbm.at[pl.ds(base, nrows), pl.ds(0, DC)],
                      table_sh.at[pl.ds(base, nrows), pl.ds(0, DC)])

    @pl.when(c == 1)
    def _():
      pltpu.sync_copy(x_hbm.at[pl.ds(base, nrows), pl.ds(DC, D_IN - DC)],
                      table_sh.at[pl.ds(base, nrows), pl.ds(0, D_IN - DC)])
      def orow(r, _):
        lane = lax.broadcasted_iota(jnp.int32, (L,), 0)
        ones_buf[r, pl.ds(0, L)] = jnp.where(lane == 0, 1.0, 0.0)
        return _
      lax.fori_loop(0, CH1, orow, 0)
      for t in range(nrows // CH1):
        pltpu.sync_copy(
            ones_buf.at[pl.ds(0, CH1), pl.ds(0, L)],
            table_sh.at[pl.ds(base + t * CH1, CH1), pl.ds(DEGC, L)])
      rem = nrows % CH1
      if rem:
        pltpu.sync_copy(
            ones_buf.at[pl.ds(0, rem), pl.ds(0, L)],
            table_sh.at[pl.ds(base + (nrows // CH1) * CH1, rem),
                        pl.ds(DEGC, L)])

  @pl.when(s < NS - 1)
  def _():
    build(RPT)

  @pl.when(s == NS - 1)
  def _():
    build(_RLAST)


NB1 = 4
ST1 = 8


@functools.lru_cache(maxsize=None)
def _seg_l1():
  @functools.partial(
      pl.kernel,
      out_type=jax.ShapeDtypeStruct((NC, NPAD, DC), jnp.float32),
      mesh=_mesh(),
      scratch_types=(
          [pltpu.VMEM((K1 // ST1, CH1), jnp.int32),
           pltpu.VMEM((K1 // ST1, CH1), jnp.int32)]
          + [pltpu.VMEM((CH1, DC), jnp.float32) for _ in range(NB1)]
          + [pltpu.SemaphoreType.DMA for _ in range(2 * NB1)]
          + [pltpu.VMEM_SHARED((NPAD, DC), jnp.float32),
             pltpu.VMEM_SHARED((N, DC), jnp.float32)]
      ),
      compiler_params=pltpu.CompilerParams(use_tc_tiling_on_sc=False),
  )
  def seg(table_hbm, src_hbm, dst_hbm, out_hbm, src_v, dst_v, *rest):
    rows = rest[:NB1]
    gsem = rest[NB1:2 * NB1]
    ssem = rest[2 * NB1:3 * NB1]
    shared = rest[3 * NB1]
    table_sh = rest[3 * NB1 + 1]
    c = lax.axis_index("c")
    s = lax.axis_index("s")

    base = s * RPT
    K1S = K1 // ST1
    for h in range(ST1):
      ig = pltpu.async_copy(src_hbm.at[s, pl.ds(h * K1S, K1S)],
                            src_v, gsem[0])
      ig2 = pltpu.async_copy(dst_hbm.at[s, pl.ds(h * K1S, K1S)],
                             dst_v, gsem[1])
      if h == 0:
        _build_table_l1(table_hbm, table_sh, rows[1], c, s)
        _zero_shared_slice(rows[0], shared, base, DC)
      ig.wait()
      ig2.wait()
      if h == 0:
        plsc.subcore_barrier()
      _pipeline(table_sh, shared, src_v, dst_v, rows, gsem, ssem, K1S)

    plsc.subcore_barrier()
    pltpu.sync_copy(shared.at[pl.ds(base, RPT)],
                    out_hbm.at[c, pl.ds(base, RPT)])

  return seg


@functools.lru_cache(maxsize=None)
def _seg_l2():
  @functools.partial(
      pl.kernel,
      out_type=jax.ShapeDtypeStruct((NC, NPAD, DW2), jnp.float32),
      mesh=_mesh(),
      scratch_types=(
          [pltpu.VMEM((K2, CH), jnp.int32),
           pltpu.VMEM((K2, CH), jnp.int32)]
          + [pltpu.VMEM((CH, DW2), jnp.float32) for _ in range(NBUF)]
          + [pltpu.SemaphoreType.DMA for _ in range(2 * NBUF)]
          + [pltpu.VMEM_SHARED((NPAD, DW2), jnp.float32),
             pltpu.VMEM_SHARED((N, DW2), jnp.float32)]
      ),
      compiler_params=pltpu.CompilerParams(use_tc_tiling_on_sc=False),
  )
  def seg(table_hbm, src_hbm, dst_hbm, out_hbm, src_v, dst_v, *rest):
    rows = rest[:NBUF]
    gsem = rest[NBUF:2 * NBUF]
    ssem = rest[2 * NBUF:3 * NBUF]
    shared = rest[3 * NBUF]
    table_sh = rest[3 * NBUF + 1]
    c = lax.axis_index("c")
    s = lax.axis_index("s")
    wid = s * NC + c

    ig = pltpu.async_copy(src_hbm.at[wid], src_v, gsem[0])
    ig2 = pltpu.async_copy(dst_hbm.at[wid], dst_v, gsem[1])
    base = s * RPT
    _load_table(table_hbm, table_sh, 0, s)
    _zero_shared_slice(rows[0], shared, base, DW2)
    ig.wait()
    ig2.wait()
    plsc.subcore_barrier()

    _pipeline(table_sh, shared, src_v, dst_v, rows, gsem, ssem, K2)

    plsc.subcore_barrier()
    pltpu.sync_copy(shared.at[pl.ds(base, RPT)],
                    out_hbm.at[c, pl.ds(base, RPT)])

  return seg


def _tc1_body(x_ref, a0_ref, a1_ref,
              w1s_ref, w1nt_ref, w1nb_ref, b1_ref, w2n_ref, w2s_ref,
              p_ref, hs_ref):
  inv = 1.0 / jnp.maximum(a1_ref[0, :, DEGC:DEGC + 1], 1.0)
  h = (jnp.dot(x_ref[...], w1s_ref[...], preferred_element_type=jnp.float32)
       + jnp.dot(a0_ref[0] * inv, w1nt_ref[...],
                 preferred_element_type=jnp.float32)
       + jnp.dot(a1_ref[0, :, :D_IN - DC] * inv, w1nb_ref[...],
                 preferred_element_type=jnp.float32)
       + b1_ref[...])
  h = jnp.maximum(h, 0.0)
  p_ref[...] = jnp.dot(h, w2n_ref[...], preferred_element_type=jnp.float32)
  hs_ref[...] = jnp.dot(h, w2s_ref[...], preferred_element_type=jnp.float32)


def _tc2_body(hs_ref, a2_ref, a1_ref, b2_ref, o_ref):
  inv = 1.0 / jnp.maximum(a1_ref[0, :, DEGC:DEGC + 1], 1.0)
  o_ref[...] = (hs_ref[...]
                + (a2_ref[0, :, :NCLS] + a2_ref[1, :, :NCLS]) * inv
                + b2_ref[...])


_R = 1000


def kernel(x, edge_index, W1_self, W1_neigh, b1, W2_self, W2_neigh, b2):
  f32 = jnp.float32
  src = edge_index[0].astype(jnp.int32)
  dst = edge_index[1].astype(jnp.int32)
  pad = EPAD - E
  src_p = jnp.concatenate([src, jnp.zeros((pad,), jnp.int32)])
  dst_pad = N + (jnp.arange(pad, dtype=jnp.int32) % (NPAD - N))
  dst_p = jnp.concatenate([dst, dst_pad])

  src_t = src_p.reshape(NS, K1, CH1)
  dst_t = dst_p.reshape(NS, K1, CH1)
  src3 = src_p.reshape(NW, K2, CH)
  dst3 = dst_p.reshape(NW, K2, CH)

  agg1 = _seg_l1()(x, src_t, dst_t)

  w1sT = W1_self.T
  w1nT_top = W1_neigh.T[:DC]
  w1nT_bot = W1_neigh.T[DC:]
  w2nT = jnp.pad(W2_neigh.T, ((0, 0), (0, DW2 - NCLS)))
  w2sT = W2_self.T
  b1r = b1.reshape(1, D_HID)
  b2r = b2.reshape(1, NCLS)

  grid = (N // _R,)
  row_spec = lambda w: pl.BlockSpec((_R, w), lambda i: (i, 0))
  full_spec = lambda a, b: pl.BlockSpec((a, b), lambda i: (0, 0))
  agg_spec = lambda h, w, j: pl.BlockSpec((1, _R, w), lambda i: (h, i, j))

  p, hs = pl.pallas_call(
      _tc1_body,
      grid=grid,
      in_specs=[
          row_spec(D_IN), agg_spec(0, DC, 0), agg_spec(1, DC, 0),
          full_spec(D_IN, D_HID), full_spec(DC, D_HID),
          full_spec(D_IN - DC, D_HID), full_spec(1, D_HID),
          full_spec(D_HID, DW2), full_spec(D_HID, NCLS),
      ],
      out_specs=[row_spec(DW2), row_spec(NCLS)],
      out_shape=[jax.ShapeDtypeStruct((N, DW2), f32),
                 jax.ShapeDtypeStruct((N, NCLS), f32)],
  )(x, agg1, agg1, w1sT, w1nT_top, w1nT_bot, b1r, w2nT, w2sT)

  agg2 = _seg_l2()(p, src3, dst3)

  out = pl.pallas_call(
      _tc2_body,
      grid=grid,
      in_specs=[
          row_spec(NCLS), pl.BlockSpec((2, _R, DW2), lambda i: (0, i, 0)),
          agg_spec(1, DC, 0), full_spec(1, NCLS),
      ],
      out_specs=row_spec(NCLS),
      out_shape=jax.ShapeDtypeStruct((N, NCLS), f32),
  )(hs, agg2, agg1, b2r)

  return out
